# Initial kernel scaffold; baseline (speedup 1.0000x reference)
import functools

import jax
import jax.numpy as jnp
from jax import lax
from jax.experimental import pallas as pl
from jax.experimental.pallas import tpu as pltpu
from jax.experimental.pallas import tpu_sc as plsc

N_NODES = 10000
N_PAD = 10240
HALF = 128
H_FEATS = 256
N_EDGES = 320000
NSUB = 16
EDGES_PER_SUB = N_EDGES // NSUB
CHUNK = 80
NCHUNK = EDGES_PER_SUB // CHUNK
ROWS_PER_SUB = N_NODES // NSUB
DEG_SLICE = N_PAD // NSUB

_sc_mesh = plsc.VectorSubcoreMesh(core_axis_name="c", subcore_axis_name="s")


@functools.partial(
    pl.kernel,
    out_type=jax.ShapeDtypeStruct((2, N_PAD), jnp.float32),
    mesh=_sc_mesh,
    scratch_types=[
        pltpu.VMEM((NCHUNK, CHUNK), jnp.int32),
        pltpu.VMEM((CHUNK,), jnp.float32),
        pltpu.VMEM((DEG_SLICE,), jnp.float32),
        pltpu.VMEM_SHARED((N_PAD,), jnp.float32),
    ],
)
def _deg_kernel(edges_hbm, deg_hbm, idx_v, ones_v, zeros_v, acc_sh):
    c = lax.axis_index("c")
    s = lax.axis_index("s")

    def fill_ones(i, carry):
        ones_v[pl.ds(i * 16, 16)] = jnp.ones((16,), jnp.float32)
        return carry

    lax.fori_loop(0, CHUNK // 16, fill_ones, 0)

    def fill_zeros(i, carry):
        zeros_v[pl.ds(i * 16, 16)] = jnp.zeros((16,), jnp.float32)
        return carry

    lax.fori_loop(0, DEG_SLICE // 16, fill_zeros, 0)

    pltpu.sync_copy(zeros_v, acc_sh.at[pl.ds(s * DEG_SLICE, DEG_SLICE)])
    pltpu.sync_copy(edges_hbm.at[c, s], idx_v)
    plsc.subcore_barrier()

    def body(i, carry):
        pltpu.sync_copy(ones_v, acc_sh.at[idx_v.at[i]], add=True)
        return carry

    lax.fori_loop(0, NCHUNK, body, 0)
    plsc.subcore_barrier()
    pltpu.sync_copy(acc_sh.at[pl.ds(s * DEG_SLICE, DEG_SLICE)],
                    deg_hbm.at[c, pl.ds(s * DEG_SLICE, DEG_SLICE)])


@functools.partial(
    pl.kernel,
    out_type=jax.ShapeDtypeStruct((2, N_NODES, HALF), jnp.float32),
    mesh=_sc_mesh,
    scratch_types=[
        pltpu.VMEM((NCHUNK, CHUNK), jnp.int32),
        pltpu.VMEM((NCHUNK, CHUNK), jnp.int32),
        pltpu.VMEM((2, CHUNK, HALF), jnp.float32),
        pltpu.VMEM_SHARED((N_NODES, HALF), jnp.float32),
        pltpu.SemaphoreType.DMA,
    ],
)
def _spmm_kernel(feat_hbm, srcidx_hbm, dstidx_hbm, zeros_hbm, out_hbm,
                 src_v, dst_v, rows_v, acc_sh, sem):
    c = lax.axis_index("c")
    s = lax.axis_index("s")
    pltpu.sync_copy(srcidx_hbm.at[c, s], src_v)
    pltpu.sync_copy(dstidx_hbm.at[s], dst_v)
    pltpu.sync_copy(zeros_hbm.at[pl.ds(s * ROWS_PER_SUB, ROWS_PER_SUB)],
                    acc_sh.at[pl.ds(s * ROWS_PER_SUB, ROWS_PER_SUB)])
    plsc.subcore_barrier()

    pltpu.async_copy(feat_hbm.at[src_v.at[0]], rows_v.at[0], sem).wait()

    def body(i, carry):
        nxt = pltpu.async_copy(feat_hbm.at[src_v.at[i + 1]],
                               rows_v.at[(i + 1) % 2], sem)
        pltpu.sync_copy(rows_v.at[i % 2], acc_sh.at[dst_v.at[i]], add=True)
        nxt.wait()
        return carry

    lax.fori_loop(0, NCHUNK - 1, body, 0)
    pltpu.sync_copy(rows_v.at[(NCHUNK - 1) % 2],
                    acc_sh.at[dst_v.at[NCHUNK - 1]], add=True)
    plsc.subcore_barrier()
    pltpu.sync_copy(acc_sh.at[pl.ds(s * ROWS_PER_SUB, ROWS_PER_SUB)],
                    out_hbm.at[c, pl.ds(s * ROWS_PER_SUB, ROWS_PER_SUB)])


def _norm(d):
    return lax.rsqrt(jnp.where(d > 0, d, 1.0))


def _mm_body(x_ref, w_ref, degs_ref, out_ref):
    ns = _norm(degs_ref[...])
    t = jnp.dot(x_ref[...], w_ref[...], preferred_element_type=jnp.float32)
    e = t * ns
    out_ref[0] = e[:, :HALF]
    out_ref[1] = e[:, HALF:]


def _scale_body(agg_ref, degs_ref, degd_ref, out_ref):
    sc = _norm(degs_ref[...]) * _norm(degd_ref[...])
    out_ref[...] = agg_ref[...] * sc[None]


def _final_body(a_ref, agg1_ref, agg2_ref, degd_ref, out_ref):
    nd = _norm(degd_ref[...])
    m = a_ref[0] * agg1_ref[...] + a_ref[1] * agg2_ref[...]
    h = m * nd[None]
    out_ref[:, :HALF] = h[0]
    out_ref[:, HALF:] = h[1]


def kernel(in_feat, edge_index, W1, alphas):
    ei = edge_index.astype(jnp.int32)
    a = jax.nn.softmax(alphas.astype(jnp.float32))

    edges4 = ei.reshape(2, NSUB, NCHUNK, CHUNK)
    deg = _deg_kernel(edges4)
    deg_src = deg[0, :N_NODES].reshape(N_NODES, 1)
    deg_dst = deg[1, :N_NODES].reshape(N_NODES, 1)

    e1 = pl.pallas_call(
        _mm_body,
        out_shape=jax.ShapeDtypeStruct((2, N_NODES, HALF), jnp.float32),
    )(in_feat, W1, deg_src)

    half_off = (jnp.arange(2, dtype=jnp.int32) * N_NODES)[:, None]
    srcidx = (ei[0][None, :] + half_off).reshape(2, NSUB, NCHUNK, CHUNK)
    dstidx = ei[1].reshape(NSUB, NCHUNK, CHUNK)
    zeros = jnp.zeros((N_NODES, HALF), jnp.float32)

    agg1 = _spmm_kernel(e1.reshape(2 * N_NODES, HALF), srcidx, dstidx, zeros)

    e2 = pl.pallas_call(
        _scale_body,
        out_shape=jax.ShapeDtypeStruct((2, N_NODES, HALF), jnp.float32),
    )(agg1, deg_src, deg_dst)

    agg2 = _spmm_kernel(e2.reshape(2 * N_NODES, HALF), srcidx, dstidx, zeros)

    res = pl.pallas_call(
        _final_body,
        out_shape=jax.ShapeDtypeStruct((N_NODES, H_FEATS), jnp.float32),
        in_specs=[
            pl.BlockSpec(memory_space=pltpu.SMEM),
            pl.BlockSpec(memory_space=pltpu.VMEM),
            pl.BlockSpec(memory_space=pltpu.VMEM),
            pl.BlockSpec(memory_space=pltpu.VMEM),
        ],
    )(a, agg1, agg2, deg_dst)
    return res


# speedup vs baseline: 5.0054x; 5.0054x over previous
import functools

import jax
import jax.numpy as jnp
from jax import lax
from jax.experimental import pallas as pl
from jax.experimental.pallas import tpu as pltpu
from jax.experimental.pallas import tpu_sc as plsc

N_NODES = 10000
N_PAD = 10240
HALF = 128
H_FEATS = 256
N_EDGES = 320000
NSUB = 16
EDGES_PER_SUB = N_EDGES // NSUB
CHUNK = 80
NCHUNK = EDGES_PER_SUB // CHUNK
ROWS_PER_SUB = N_PAD // NSUB
DEG_SLICE = N_PAD // NSUB
NQ = 4
QF = H_FEATS // NQ

_sc_mesh = plsc.VectorSubcoreMesh(core_axis_name="c", subcore_axis_name="s")
_sc_params = pltpu.CompilerParams(use_tc_tiling_on_sc=False)


@functools.partial(
    pl.kernel,
    out_type=jax.ShapeDtypeStruct((2, N_PAD), jnp.float32),
    mesh=_sc_mesh,
    scratch_types=[
        pltpu.VMEM((NCHUNK, CHUNK), jnp.int32),
        pltpu.VMEM((CHUNK,), jnp.float32),
        pltpu.VMEM((DEG_SLICE,), jnp.float32),
        pltpu.VMEM_SHARED((N_PAD,), jnp.float32),
    ],
    compiler_params=_sc_params,
)
def _deg_kernel(edges_hbm, deg_hbm, idx_v, ones_v, zeros_v, acc_sh):
    c = lax.axis_index("c")
    s = lax.axis_index("s")

    def fill_ones(i, carry):
        ones_v[pl.ds(i * 16, 16)] = jnp.ones((16,), jnp.float32)
        return carry

    lax.fori_loop(0, CHUNK // 16, fill_ones, 0)

    def fill_zeros(i, carry):
        zeros_v[pl.ds(i * 16, 16)] = jnp.zeros((16,), jnp.float32)
        return carry

    lax.fori_loop(0, DEG_SLICE // 16, fill_zeros, 0)

    pltpu.sync_copy(zeros_v, acc_sh.at[pl.ds(s * DEG_SLICE, DEG_SLICE)])
    pltpu.sync_copy(edges_hbm.at[c, s], idx_v)
    plsc.subcore_barrier()

    def body(i, carry):
        pltpu.sync_copy(ones_v, acc_sh.at[idx_v.at[i]], add=True)
        return carry

    lax.fori_loop(0, NCHUNK, body, 0)
    plsc.subcore_barrier()
    pltpu.sync_copy(acc_sh.at[pl.ds(s * DEG_SLICE, DEG_SLICE)],
                    deg_hbm.at[c, pl.ds(s * DEG_SLICE, DEG_SLICE)])


@functools.partial(
    pl.kernel,
    out_type=jax.ShapeDtypeStruct((NQ, N_PAD, QF), jnp.float32),
    mesh=_sc_mesh,
    scratch_types=[
        pltpu.VMEM((2, NCHUNK, CHUNK), jnp.int32),
        pltpu.VMEM((NCHUNK, CHUNK), jnp.int32),
        pltpu.VMEM((2, CHUNK, QF), jnp.float32),
        pltpu.VMEM_SHARED((N_PAD, QF), jnp.float32),
        pltpu.SemaphoreType.DMA,
    ],
    compiler_params=_sc_params,
)
def _spmm_kernel(feat_hbm, srcidx_hbm, dstidx_hbm, zeros_hbm, out_hbm,
                 src_v, dst_v, rows_v, acc_sh, sem):
    c = lax.axis_index("c")
    s = lax.axis_index("s")
    pltpu.sync_copy(srcidx_hbm.at[c * 2, s], src_v.at[0])
    pltpu.sync_copy(srcidx_hbm.at[c * 2 + 1, s], src_v.at[1])
    pltpu.sync_copy(dstidx_hbm.at[s], dst_v)

    for q in range(2):
        sq = src_v.at[q]
        pltpu.sync_copy(zeros_hbm.at[pl.ds(s * ROWS_PER_SUB, ROWS_PER_SUB)],
                        acc_sh.at[pl.ds(s * ROWS_PER_SUB, ROWS_PER_SUB)])
        plsc.subcore_barrier()

        pltpu.async_copy(feat_hbm.at[sq.at[0]], rows_v.at[0], sem).wait()

        def body(i, carry, sq=sq):
            nxt = pltpu.async_copy(feat_hbm.at[sq.at[i + 1]],
                                   rows_v.at[(i + 1) % 2], sem)
            pltpu.sync_copy(rows_v.at[i % 2], acc_sh.at[dst_v.at[i]], add=True)
            nxt.wait()
            return carry

        lax.fori_loop(0, NCHUNK - 1, body, 0)
        pltpu.sync_copy(rows_v.at[(NCHUNK - 1) % 2],
                        acc_sh.at[dst_v.at[NCHUNK - 1]], add=True)
        plsc.subcore_barrier()
        pltpu.sync_copy(
            acc_sh.at[pl.ds(s * ROWS_PER_SUB, ROWS_PER_SUB)],
            out_hbm.at[c * 2 + q, pl.ds(s * ROWS_PER_SUB, ROWS_PER_SUB)])


def _norm(d):
    return lax.rsqrt(jnp.where(d > 0, d, 1.0))


def _mm_body(x_ref, w_ref, degs_ref, out_ref):
    ns = _norm(degs_ref[...])
    t = jnp.dot(x_ref[...], w_ref[...], preferred_element_type=jnp.float32)
    e = t * ns
    for q in range(NQ):
        out_ref[q] = e[:, q * QF:(q + 1) * QF]


def _scale_body(agg_ref, degs_ref, degd_ref, out_ref):
    sc = _norm(degs_ref[...]) * _norm(degd_ref[...])
    out_ref[...] = agg_ref[...] * sc[None]


def _final_body(a_ref, agg1_ref, agg2_ref, degd_ref, out_ref):
    nd = _norm(degd_ref[...])
    m = a_ref[0] * agg1_ref[...] + a_ref[1] * agg2_ref[...]
    h = m * nd[None]
    for q in range(NQ):
        out_ref[:, q * QF:(q + 1) * QF] = h[q]


_RB = 2000
_RBP = 2048


def kernel(in_feat, edge_index, W1, alphas):
    ei = edge_index.astype(jnp.int32)
    a = jax.nn.softmax(alphas.astype(jnp.float32))

    edges4 = ei.reshape(2, NSUB, NCHUNK, CHUNK)
    deg = _deg_kernel(edges4)
    deg_src = deg[0].reshape(N_PAD, 1)
    deg_dst = deg[1].reshape(N_PAD, 1)

    e1 = pl.pallas_call(
        _mm_body,
        grid=(N_NODES // _RB,),
        in_specs=[
            pl.BlockSpec((_RB, 128), lambda i: (i, 0)),
            pl.BlockSpec((128, H_FEATS), lambda i: (0, 0)),
            pl.BlockSpec((_RB, 1), lambda i: (i, 0)),
        ],
        out_specs=pl.BlockSpec((NQ, _RB, QF), lambda i: (0, i, 0)),
        out_shape=jax.ShapeDtypeStruct((NQ, N_PAD, QF), jnp.float32),
    )(in_feat, W1, deg_src)

    q_off = (jnp.arange(NQ, dtype=jnp.int32) * N_PAD)[:, None]
    srcidx = (ei[0][None, :] + q_off).reshape(NQ, NSUB, NCHUNK, CHUNK)
    dstidx = ei[1].reshape(NSUB, NCHUNK, CHUNK)
    zeros = jnp.zeros((N_PAD, QF), jnp.float32)

    agg1 = _spmm_kernel(e1.reshape(NQ * N_PAD, QF), srcidx, dstidx, zeros)

    e2 = pl.pallas_call(
        _scale_body,
        grid=(N_PAD // _RBP,),
        in_specs=[
            pl.BlockSpec((NQ, _RBP, QF), lambda i: (0, i, 0)),
            pl.BlockSpec((_RBP, 1), lambda i: (i, 0)),
            pl.BlockSpec((_RBP, 1), lambda i: (i, 0)),
        ],
        out_specs=pl.BlockSpec((NQ, _RBP, QF), lambda i: (0, i, 0)),
        out_shape=jax.ShapeDtypeStruct((NQ, N_PAD, QF), jnp.float32),
    )(agg1, deg_src, deg_dst)

    agg2 = _spmm_kernel(e2.reshape(NQ * N_PAD, QF), srcidx, dstidx, zeros)

    res = pl.pallas_call(
        _final_body,
        grid=(N_NODES // _RB,),
        in_specs=[
            pl.BlockSpec(memory_space=pltpu.SMEM),
            pl.BlockSpec((NQ, _RB, QF), lambda i: (0, i, 0)),
            pl.BlockSpec((NQ, _RB, QF), lambda i: (0, i, 0)),
            pl.BlockSpec((_RB, 1), lambda i: (i, 0)),
        ],
        out_specs=pl.BlockSpec((_RB, H_FEATS), lambda i: (i, 0)),
        out_shape=jax.ShapeDtypeStruct((N_NODES, H_FEATS), jnp.float32),
    )(a, agg1, agg2, deg_dst)
    return res
